# baseline (speedup 1.0000x reference)
import functools

import jax
import jax.numpy as jnp
import numpy as np
from jax import lax
from jax.experimental import pallas as pl
from jax.experimental.pallas import tpu as pltpu
import jax.experimental.pallas.tpu_sc as plsc

R = 64
N = 100000
NC = 2
NS = 16
NW = NC * NS
ROWS_PER_W = R // NW
W = 10000
NWIN = N // W
VPW = W // 16
UNROLL = 5
NBINS = 1 << 16
L1 = NBINS // 16
L2 = L1 // 16

_U = jnp.uint32
_SIGN = np.uint32(0x80000000)
_POSM = np.uint32(0x7FFFFFFF)
_ZERO_U = np.uint32(0)


def _key_from_bits(u):
    u = jnp.where(u == _SIGN, _ZERO_U, u)
    mask = jnp.where(u >= _SIGN, _ZERO_U, _POSM)
    return u ^ mask


def _zero_hist(hist):
    zeros = lax.iota(jnp.int32, 16) * 0

    def body(i, _):
        for j in range(16):
            hist[pl.ds((i * 16 + j) * 16, 16)] = zeros
        return 0

    lax.fori_loop(0, L1 // 16, body, 0)


def _prefix_sum(hist, t0, t1):
    iota = lax.iota(jnp.int32, 16)

    def l0(g, _):
        acc = iota * 0
        for j in range(16):
            v = hist[pl.ds((g * 16 + j) * 16, 16)]
            acc = jnp.where(iota == j, jnp.sum(v), acc)
        t0[pl.ds(g * 16, 16)] = acc
        return 0

    lax.fori_loop(0, L1 // 16, l0, 0)

    def l1(g, _):
        acc = iota * 0
        for j in range(16):
            v = t0[pl.ds((g * 16 + j) * 16, 16)]
            acc = jnp.where(iota == j, jnp.sum(v), acc)
        t1[pl.ds(g * 16, 16)] = acc
        return 0

    lax.fori_loop(0, L2 // 16, l1, 0)

    def l2(i, c):
        v = t1[pl.ds(i * 16, 16)]
        s = plsc.cumsum(v)
        t1[pl.ds(i * 16, 16)] = s - v + c
        return c + jnp.sum(v)

    lax.fori_loop(0, L2 // 16, l2, jnp.int32(0))

    def l1b(g, _):
        tv = t1[pl.ds(g * 16, 16)]
        for j in range(16):
            i = g * 16 + j
            v = t0[pl.ds(i * 16, 16)]
            s = plsc.cumsum(v)
            t0[pl.ds(i * 16, 16)] = s - v + tv[j]
        return 0

    lax.fori_loop(0, L2 // 16, l1b, 0)

    def l0b(g, _):
        tv = t0[pl.ds(g * 16, 16)]
        for j in range(16):
            i = g * 16 + j
            v = hist[pl.ds(i * 16, 16)]
            s = plsc.cumsum(v)
            hist[pl.ds(i * 16, 16)] = s - v + tv[j]
        return 0

    lax.fori_loop(0, L1 // 16, l0b, 0)


def _worker_id():
    return lax.axis_index("s") * NC + lax.axis_index("c")


def _radix_pass(in_val_hbm, digit_fn, payload_fn, out_a_fn, out_a_hbm,
                out_b_hbm, load_b_fn,
                hist, t0, t1, sbuf, ibuf, pbuf, kbuf, sem_out, rbase):
    def hist_win(w, _):
        base = pl.multiple_of(rbase + w * W, 8)
        pltpu.sync_copy(in_val_hbm.at[pl.ds(base, W)], sbuf)

        def vreg(jj, _):
            for u_ in range(UNROLL):
                j = jj * UNROLL + u_
                _, d = digit_fn(sbuf[pl.ds(j * 16, 16)])
                cnt, last = plsc.scan_count(d)
                plsc.addupdate_scatter(hist, [d], cnt, mask=last)
            return 0

        lax.fori_loop(0, VPW // UNROLL, vreg, 0)
        return 0

    with jax.named_scope("hist_phase"):
        lax.fori_loop(0, NWIN, hist_win, 0)
    with jax.named_scope("prefix_phase"):
        _prefix_sum(hist, t0, t1)
    iota = lax.iota(jnp.int32, 16)

    def perm_win(w, _):
        base = pl.multiple_of(rbase + w * W, 8)
        pltpu.sync_copy(in_val_hbm.at[pl.ds(base, W)], sbuf)
        if payload_fn is None:
            load_b_fn(base, ibuf)

        def vreg(jj, _):
            for u_ in range(UNROLL):
                j = jj * UNROLL + u_
                kk, d = digit_fn(sbuf[pl.ds(j * 16, 16)])
                cnt, last = plsc.scan_count(d)
                bse = plsc.load_gather(hist, [d])
                pos = bse + cnt - 1
                plsc.store_scatter(hist, [d], pos + 1, mask=last)
                pbuf[pl.ds(j * 16, 16)] = pos + rbase
                kbuf[pl.ds(j * 16, 16)] = out_a_fn(kk)
                if payload_fn is not None:
                    ibuf[pl.ds(j * 16, 16)] = payload_fn(w, j, iota)
            return 0

        lax.fori_loop(0, VPW // UNROLL, vreg, 0)
        with jax.named_scope("perm_scatter"):
            pltpu.async_copy(kbuf, out_a_hbm.at[pbuf], sem_out).wait()
            pltpu.async_copy(ibuf, out_b_hbm.at[pbuf], sem_out).wait()
        return 0

    with jax.named_scope("perm_phase"):
        lax.fori_loop(0, NWIN, perm_win, 0)


def _digit_lo(x_f32vec):
    u = plsc.bitcast(x_f32vec, _U)
    kk = _key_from_bits(u)
    return kk, (kk & np.uint32(0xFFFF)).astype(jnp.int32)


def _digit_hi(x_f32vec):
    kk = plsc.bitcast(x_f32vec, _U)
    return kk, (kk >> np.uint32(16)).astype(jnp.int32)


def _pass0_body(scores_hbm, keys_hbm, idxs_hbm,
                hist, t0, t1, sbuf, ibuf, pbuf, kbuf, sem_out):
    wid = _worker_id()

    def do_row(row_i, _):
        rbase = pl.multiple_of((wid * ROWS_PER_W + row_i) * N, 8)
        _zero_hist(hist)
        _radix_pass(
            scores_hbm, _digit_lo,
            payload_fn=lambda w, j, iota: w * W + j * 16 + iota,
            out_a_fn=lambda kk: plsc.bitcast(kk, jnp.float32),
            out_a_hbm=keys_hbm, out_b_hbm=idxs_hbm, load_b_fn=None,
            hist=hist, t0=t0, t1=t1, sbuf=sbuf, ibuf=ibuf, pbuf=pbuf,
            kbuf=kbuf, sem_out=sem_out, rbase=rbase)
        return 0

    lax.fori_loop(0, ROWS_PER_W, do_row, 0)


def _pass1_body(keys_hbm, idxs_hbm, probs_hbm, words_hbm,
                hist, t0, t1, sbuf, ibuf, pbuf, kbuf, sem_out):
    wid = _worker_id()

    def inv_key(kk):
        mask = jnp.where(kk >= _SIGN, _ZERO_U, _POSM)
        return plsc.bitcast(kk ^ mask, jnp.float32)

    def load_idx(base, dst):
        pltpu.sync_copy(idxs_hbm.at[pl.ds(base, W)], dst)

    def do_row(row_i, _):
        rbase = pl.multiple_of((wid * ROWS_PER_W + row_i) * N, 8)
        _zero_hist(hist)
        _radix_pass(
            keys_hbm, _digit_hi,
            payload_fn=None,
            out_a_fn=inv_key,
            out_a_hbm=probs_hbm, out_b_hbm=words_hbm, load_b_fn=load_idx,
            hist=hist, t0=t0, t1=t1, sbuf=sbuf, ibuf=ibuf, pbuf=pbuf,
            kbuf=kbuf, sem_out=sem_out, rbase=rbase)
        return 0

    lax.fori_loop(0, ROWS_PER_W, do_row, 0)


def _make_kernel(body, out_dtypes):
    mesh = plsc.VectorSubcoreMesh(core_axis_name="c", subcore_axis_name="s")
    return functools.partial(
        pl.kernel,
        out_type=[jax.ShapeDtypeStruct((R * N,), dt) for dt in out_dtypes],
        mesh=mesh,
        scratch_types=[
            pltpu.VMEM((NBINS,), jnp.int32),
            pltpu.VMEM((L1,), jnp.int32),
            pltpu.VMEM((L2,), jnp.int32),
            pltpu.VMEM((W,), jnp.float32),
            pltpu.VMEM((W,), jnp.int32),
            pltpu.VMEM((W,), jnp.int32),
            pltpu.VMEM((W,), jnp.float32),
            pltpu.SemaphoreType.DMA,
        ],
        compiler_params=pltpu.CompilerParams(needs_layout_passes=False),
    )(body)


def kernel(scores, k):
    del k
    keys, idxs = _make_kernel(_pass0_body, (jnp.float32, jnp.int32))(
        scores.reshape(-1))
    probs, words = _make_kernel(_pass1_body, (jnp.float32, jnp.int32))(
        keys, idxs)
    return probs.reshape(R, N), words.reshape(R, N)


# speedup vs baseline: 5.0359x; 5.0359x over previous
import functools

import jax
import jax.numpy as jnp
import numpy as np
from jax import lax
from jax.experimental import pallas as pl
from jax.experimental.pallas import tpu as pltpu
import jax.experimental.pallas.tpu_sc as plsc

R = 64
N = 100000
NC = 2
NS = 16
NW = NC * NS
ROWS_PER_W = R // NW
W = 4000
NWIN = N // W
VPW = W // 16
UNROLL = 10
NQ = 5
QH = N // NQ
CW = 4000
CWIN = QH // CW
NBINS = 1 << 16
L1 = NBINS // 16
L2 = L1 // 16

_U = jnp.uint32
_SIGN = np.uint32(0x80000000)
_POSM = np.uint32(0x7FFFFFFF)
_ZERO_U = np.uint32(0)


def _key_from_bits(u):
    u = jnp.where(u == _SIGN, _ZERO_U, u)
    mask = jnp.where(u >= _SIGN, _ZERO_U, _POSM)
    return u ^ mask


def _zero_hist(hist):
    zeros = lax.iota(jnp.int32, 16) * 0

    def body(i, _):
        for j in range(16):
            hist[pl.ds((i * 16 + j) * 16, 16)] = zeros
        return 0

    lax.fori_loop(0, L1 // 16, body, 0)


def _prefix_sum(hist, t0, t1):
    iota = lax.iota(jnp.int32, 16)

    def l0(g, _):
        acc = iota * 0
        for j in range(16):
            v = hist[pl.ds((g * 16 + j) * 16, 16)]
            acc = jnp.where(iota == j, jnp.sum(v), acc)
        t0[pl.ds(g * 16, 16)] = acc
        return 0

    lax.fori_loop(0, L1 // 16, l0, 0)

    def l1(g, _):
        acc = iota * 0
        for j in range(16):
            v = t0[pl.ds((g * 16 + j) * 16, 16)]
            acc = jnp.where(iota == j, jnp.sum(v), acc)
        t1[pl.ds(g * 16, 16)] = acc
        return 0

    lax.fori_loop(0, L2 // 16, l1, 0)

    def l2(i, c):
        v = t1[pl.ds(i * 16, 16)]
        s = plsc.cumsum(v)
        t1[pl.ds(i * 16, 16)] = s - v + c
        return c + jnp.sum(v)

    lax.fori_loop(0, L2 // 16, l2, jnp.int32(0))

    def l1b(g, _):
        tv = t1[pl.ds(g * 16, 16)]
        for j in range(16):
            i = g * 16 + j
            v = t0[pl.ds(i * 16, 16)]
            s = plsc.cumsum(v)
            t0[pl.ds(i * 16, 16)] = s - v + tv[j]
        return 0

    lax.fori_loop(0, L2 // 16, l1b, 0)

    def l0b(g, _):
        tv = t0[pl.ds(g * 16, 16)]
        for j in range(16):
            i = g * 16 + j
            v = hist[pl.ds(i * 16, 16)]
            s = plsc.cumsum(v)
            hist[pl.ds(i * 16, 16)] = s - v + tv[j]
        return 0

    lax.fori_loop(0, L1 // 16, l0b, 0)


def _digit_lo(x_i32vec):
    u = plsc.bitcast(x_i32vec, _U)
    kk = _key_from_bits(u)
    return kk, (kk & np.uint32(0xFFFF)).astype(jnp.int32)


def _digit_hi(x_i32vec):
    kk = plsc.bitcast(x_i32vec, _U)
    return kk, (kk >> np.uint32(16)).astype(jnp.int32)


def _export_quarter(spm, sbase, out_hbm, rbase, q, stage):
    def chunk(w, _):
        st = stage.at[pl.ds(0, CW)]
        pltpu.sync_copy(spm.at[pl.ds(sbase + w * CW, CW)], st)
        pltpu.sync_copy(st, out_hbm.at[pl.ds(rbase + q * QH + w * CW, CW)])
        return 0

    lax.fori_loop(0, CWIN, chunk, 0)


def _quarter_idx(pos, q, sbase):
    local = pos - q * QH
    return jnp.where((local >= 0) & (local < QH), local + sbase,
                     jnp.int32(-1))


def _radix_pass(in_hbm, digit_fn, is_pass0, out_a_fn,
                out_a_hbm, out_b_hbm, idx_in_hbm, pos_hbm,
                spm, hist, t0, t1, sbuf, ibuf, pbuf, pbufs, kbuf,
                sem_out, rbase, sid):
    sbase = pl.multiple_of(sid * QH, 8)

    def hist_win(w, _):
        base = pl.multiple_of(rbase + w * W, 8)
        pltpu.sync_copy(in_hbm.at[pl.ds(base, W)], sbuf)

        def vreg(jj, _):
            for u_ in range(UNROLL):
                j = jj * UNROLL + u_
                _, d = digit_fn(sbuf[pl.ds(j * 16, 16)])
                cnt, last = plsc.scan_count(d)
                plsc.addupdate_scatter(hist, [d], cnt, mask=last)
            return 0

        lax.fori_loop(0, VPW // UNROLL, vreg, 0)
        return 0

    lax.fori_loop(0, NWIN, hist_win, 0)
    _prefix_sum(hist, t0, t1)

    def perm_win_a0(w, _):
        base = pl.multiple_of(rbase + w * W, 8)
        pltpu.sync_copy(in_hbm.at[pl.ds(base, W)], sbuf)

        def vreg(jj, _):
            for u_ in range(UNROLL):
                j = jj * UNROLL + u_
                kk, d = digit_fn(sbuf[pl.ds(j * 16, 16)])
                cnt, last = plsc.scan_count(d)
                bse = plsc.load_gather(hist, [d])
                pos = bse + cnt - 1
                plsc.store_scatter(hist, [d], pos + 1, mask=last)
                pbuf[pl.ds(j * 16, 16)] = pos
                pbufs[pl.ds(j * 16, 16)] = _quarter_idx(pos, 0, sbase)
                kbuf[pl.ds(j * 16, 16)] = out_a_fn(kk)
            return 0

        lax.fori_loop(0, VPW // UNROLL, vreg, 0)
        pltpu.async_copy(kbuf, spm.at[plsc.Indices(pbufs, ignored_value=-1)],
                         sem_out).wait()
        pltpu.sync_copy(pbuf, pos_hbm.at[pl.ds(base, W)])
        return 0

    lax.fori_loop(0, NWIN, perm_win_a0, 0)
    plsc.subcore_barrier()
    _export_quarter(spm, sbase, out_a_hbm, rbase, 0, kbuf)

    def perm_win_a(w, q):
        base = pl.multiple_of(rbase + w * W, 8)
        pltpu.sync_copy(in_hbm.at[pl.ds(base, W)], sbuf)
        pltpu.sync_copy(pos_hbm.at[pl.ds(base, W)], pbuf)

        def vreg(jj, _):
            for u_ in range(UNROLL):
                j = jj * UNROLL + u_
                kk, _ = digit_fn(sbuf[pl.ds(j * 16, 16)])
                pos = pbuf[pl.ds(j * 16, 16)]
                pbufs[pl.ds(j * 16, 16)] = _quarter_idx(pos, q, sbase)
                kbuf[pl.ds(j * 16, 16)] = out_a_fn(kk)
            return 0

        lax.fori_loop(0, VPW // UNROLL, vreg, 0)
        pltpu.async_copy(kbuf, spm.at[plsc.Indices(pbufs, ignored_value=-1)],
                         sem_out).wait()
        return 0

    def a_round(q, _):
        lax.fori_loop(0, NWIN, lambda w, __: perm_win_a(w, q), 0)
        plsc.subcore_barrier()
        _export_quarter(spm, sbase, out_a_hbm, rbase, q, kbuf)
        return 0

    lax.fori_loop(1, NQ, a_round, 0)

    iota = lax.iota(jnp.int32, 16)

    def payload_win(w, q):
        base = pl.multiple_of(rbase + w * W, 8)
        pltpu.sync_copy(pos_hbm.at[pl.ds(base, W)], pbuf)
        if not is_pass0:
            pltpu.sync_copy(idx_in_hbm.at[pl.ds(base, W)], ibuf)

        def vreg(jj, _):
            for u_ in range(UNROLL):
                j = jj * UNROLL + u_
                pos = pbuf[pl.ds(j * 16, 16)]
                pbufs[pl.ds(j * 16, 16)] = _quarter_idx(pos, q, sbase)
                if is_pass0:
                    ibuf[pl.ds(j * 16, 16)] = w * W + j * 16 + iota
            return 0

        lax.fori_loop(0, VPW // UNROLL, vreg, 0)
        pltpu.async_copy(ibuf, spm.at[plsc.Indices(pbufs, ignored_value=-1)],
                         sem_out).wait()
        return 0

    def b_round(q, _):
        lax.fori_loop(0, NWIN, lambda w, __: payload_win(w, q), 0)
        plsc.subcore_barrier()
        _export_quarter(spm, sbase, out_b_hbm, rbase, q, kbuf)
        return 0

    lax.fori_loop(0, NQ, b_round, 0)


def _key_out_fn(kk):
    return plsc.bitcast(kk, jnp.int32)


def _prob_out_fn(kk):
    mask = jnp.where(kk >= _SIGN, _ZERO_U, _POSM)
    return plsc.bitcast(kk ^ mask, jnp.int32)


def _run_rows(in_hbm, digit_fn, is_pass0, out_a_fn, out_a, out_b, idx_in,
              pos_hbm, spm, hist, t0, t1, sbuf, ibuf, pbuf, pbufs, kbuf,
              sem_out):
    cid = lax.axis_index("c")
    sid = lax.axis_index("s")
    wid = sid * NC + cid

    def do_row(row_i, _):
        rbase = pl.multiple_of((wid * ROWS_PER_W + row_i) * N, 8)
        _zero_hist(hist)
        _radix_pass(
            in_hbm, digit_fn, is_pass0, out_a_fn, out_a, out_b, idx_in,
            pos_hbm, spm, hist, t0, t1, sbuf, ibuf, pbuf, pbufs, kbuf,
            sem_out, rbase, sid)
        return 0

    lax.fori_loop(0, ROWS_PER_W, do_row, 0)


def _pass0_body(scores, keys_o, idxs_o, pos_o,
                spm, hist, t0, t1, sbuf, ibuf, pbuf, pbufs, kbuf, sem_out):
    _run_rows(scores, _digit_lo, True, _key_out_fn, keys_o, idxs_o, None,
              pos_o, spm, hist, t0, t1, sbuf, ibuf, pbuf, pbufs, kbuf,
              sem_out)


def _pass1_body(keys_i, idxs_i, probs_o, words_o, pos_o,
                spm, hist, t0, t1, sbuf, ibuf, pbuf, pbufs, kbuf, sem_out):
    _run_rows(keys_i, _digit_hi, False, _prob_out_fn, probs_o, words_o,
              idxs_i, pos_o, spm, hist, t0, t1, sbuf, ibuf, pbuf, pbufs,
              kbuf, sem_out)


def _make_kernel(is_pass0):
    mesh = plsc.VectorSubcoreMesh(core_axis_name="c", subcore_axis_name="s")
    return functools.partial(
        pl.kernel,
        out_type=[jax.ShapeDtypeStruct((R * N,), jnp.int32)
                  for _ in range(3)],
        mesh=mesh,
        scratch_types=[
            pltpu.VMEM_SHARED((NS * QH,), jnp.int32),
            pltpu.VMEM((NBINS,), jnp.int32),
            pltpu.VMEM((L1,), jnp.int32),
            pltpu.VMEM((L2,), jnp.int32),
            pltpu.VMEM((W,), jnp.int32),
            pltpu.VMEM((W,), jnp.int32),
            pltpu.VMEM((W,), jnp.int32),
            pltpu.VMEM((W,), jnp.int32),
            pltpu.VMEM((W,), jnp.int32),
            pltpu.SemaphoreType.DMA,
        ],
        compiler_params=pltpu.CompilerParams(needs_layout_passes=False),
    )(_pass0_body if is_pass0 else _pass1_body)


def kernel(scores, k):
    del k
    s_i32 = lax.bitcast_convert_type(scores, jnp.int32).reshape(-1)
    keys, idxs, _ = _make_kernel(True)(s_i32)
    probs_i32, words, _ = _make_kernel(False)(keys, idxs)
    probs = lax.bitcast_convert_type(probs_i32.reshape(R, N), jnp.float32)
    return probs, words.reshape(R, N)


# speedup vs baseline: 8.8760x; 1.7625x over previous
import functools

import jax
import jax.numpy as jnp
import numpy as np
from jax import lax
from jax.experimental import pallas as pl
from jax.experimental.pallas import tpu as pltpu
import jax.experimental.pallas.tpu_sc as plsc

R = 64
N = 100000
NC = 2
NS = 16
NW = NC * NS
ROWS_PER_W = R // NW
W = 2000
NWIN = N // W
VPW = W // 16
UNROLL = 5
NQ = 2
QH = N // NQ
CW = 2000
CWIN = QH // CW
NBINS = 1 << 16
L1 = NBINS // 16
L2 = L1 // 16

_U = jnp.uint32
_SIGN = np.uint32(0x80000000)
_POSM = np.uint32(0x7FFFFFFF)
_ZERO_U = np.uint32(0)


def _key_from_bits(u):
    u = jnp.where(u == _SIGN, _ZERO_U, u)
    mask = jnp.where(u >= _SIGN, _ZERO_U, _POSM)
    return u ^ mask


def _zero_hist(hist):
    zeros = lax.iota(jnp.int32, 16) * 0

    def body(i, _):
        for j in range(16):
            hist[pl.ds((i * 16 + j) * 16, 16)] = zeros
        return 0

    lax.fori_loop(0, L1 // 16, body, 0)


def _prefix_sum(hist, t0, t1):
    iota = lax.iota(jnp.int32, 16)

    def l0(g, _):
        acc = iota * 0
        for j in range(16):
            v = hist[pl.ds((g * 16 + j) * 16, 16)]
            acc = jnp.where(iota == j, jnp.sum(v), acc)
        t0[pl.ds(g * 16, 16)] = acc
        return 0

    lax.fori_loop(0, L1 // 16, l0, 0)

    def l1(g, _):
        acc = iota * 0
        for j in range(16):
            v = t0[pl.ds((g * 16 + j) * 16, 16)]
            acc = jnp.where(iota == j, jnp.sum(v), acc)
        t1[pl.ds(g * 16, 16)] = acc
        return 0

    lax.fori_loop(0, L2 // 16, l1, 0)

    def l2(i, c):
        v = t1[pl.ds(i * 16, 16)]
        s = plsc.cumsum(v)
        t1[pl.ds(i * 16, 16)] = s - v + c
        return c + jnp.sum(v)

    lax.fori_loop(0, L2 // 16, l2, jnp.int32(0))

    def l1b(g, _):
        tv = t1[pl.ds(g * 16, 16)]
        for j in range(16):
            i = g * 16 + j
            v = t0[pl.ds(i * 16, 16)]
            s = plsc.cumsum(v)
            t0[pl.ds(i * 16, 16)] = s - v + tv[j]
        return 0

    lax.fori_loop(0, L2 // 16, l1b, 0)

    def l0b(g, _):
        tv = t0[pl.ds(g * 16, 16)]
        for j in range(16):
            i = g * 16 + j
            v = hist[pl.ds(i * 16, 16)]
            s = plsc.cumsum(v)
            hist[pl.ds(i * 16, 16)] = s - v + tv[j]
        return 0

    lax.fori_loop(0, L1 // 16, l0b, 0)


def _digit_lo(x_i32vec):
    u = plsc.bitcast(x_i32vec, _U)
    kk = _key_from_bits(u)
    return kk, (kk & np.uint32(0xFFFF)).astype(jnp.int32)


def _digit_hi(x_i32vec):
    kk = plsc.bitcast(x_i32vec, _U)
    return kk, (kk >> np.uint32(16)).astype(jnp.int32)


def _export_quarter(spm, sbase, out_hbm, rbase, q, stage):
    def chunk(w, _):
        st = stage.at[pl.ds(0, CW)]
        pltpu.sync_copy(spm.at[pl.ds(sbase + w * CW, CW)], st)
        pltpu.sync_copy(st, out_hbm.at[pl.ds(rbase + q * QH + w * CW, CW)])
        return 0

    lax.fori_loop(0, CWIN, chunk, 0)


def _quarter_idx(pos, q, sbase):
    local = pos - q * QH
    return jnp.where((local >= 0) & (local < QH), local + sbase,
                     jnp.int32(-1))


def _radix_pass(in_hbm, digit_fn, is_pass0, out_a_fn,
                out_a_hbm, out_b_hbm, idx_in_hbm, pos_hbm,
                spm, hist, t0, t1, sbuf, ibuf, pbuf, pbufs, kbuf,
                sem_out, rbase, sid):
    sbase = pl.multiple_of(sid * QH, 8)

    def hist_win(w, _):
        base = pl.multiple_of(rbase + w * W, 8)
        pltpu.sync_copy(in_hbm.at[pl.ds(base, W)], sbuf)

        def vreg(jj, _):
            for u_ in range(UNROLL):
                j = jj * UNROLL + u_
                _, d = digit_fn(sbuf[pl.ds(j * 16, 16)])
                cnt, last = plsc.scan_count(d)
                plsc.addupdate_scatter(hist, [d], cnt, mask=last)
            return 0

        lax.fori_loop(0, VPW // UNROLL, vreg, 0)
        return 0

    lax.fori_loop(0, NWIN, hist_win, 0)
    _prefix_sum(hist, t0, t1)

    def perm_win_a0(w, _):
        base = pl.multiple_of(rbase + w * W, 8)
        pltpu.sync_copy(in_hbm.at[pl.ds(base, W)], sbuf)

        def vreg(jj, _):
            for u_ in range(UNROLL):
                j = jj * UNROLL + u_
                kk, d = digit_fn(sbuf[pl.ds(j * 16, 16)])
                cnt, last = plsc.scan_count(d)
                bse = plsc.load_gather(hist, [d])
                pos = bse + cnt - 1
                plsc.store_scatter(hist, [d], pos + 1, mask=last)
                pbuf[pl.ds(j * 16, 16)] = pos
                pbufs[pl.ds(j * 16, 16)] = _quarter_idx(pos, 0, sbase)
                kbuf[pl.ds(j * 16, 16)] = out_a_fn(kk)
            return 0

        lax.fori_loop(0, VPW // UNROLL, vreg, 0)
        pltpu.async_copy(kbuf, spm.at[plsc.Indices(pbufs, ignored_value=-1)],
                         sem_out).wait()
        pltpu.sync_copy(pbuf, pos_hbm.at[pl.ds(base, W)])
        return 0

    lax.fori_loop(0, NWIN, perm_win_a0, 0)
    plsc.subcore_barrier()
    _export_quarter(spm, sbase, out_a_hbm, rbase, 0, kbuf)

    def perm_win_a(w, q):
        base = pl.multiple_of(rbase + w * W, 8)
        pltpu.sync_copy(in_hbm.at[pl.ds(base, W)], sbuf)
        pltpu.sync_copy(pos_hbm.at[pl.ds(base, W)], pbuf)

        def vreg(jj, _):
            for u_ in range(UNROLL):
                j = jj * UNROLL + u_
                kk, _ = digit_fn(sbuf[pl.ds(j * 16, 16)])
                pos = pbuf[pl.ds(j * 16, 16)]
                pbufs[pl.ds(j * 16, 16)] = _quarter_idx(pos, q, sbase)
                kbuf[pl.ds(j * 16, 16)] = out_a_fn(kk)
            return 0

        lax.fori_loop(0, VPW // UNROLL, vreg, 0)
        pltpu.async_copy(kbuf, spm.at[plsc.Indices(pbufs, ignored_value=-1)],
                         sem_out).wait()
        return 0

    def a_round(q, _):
        lax.fori_loop(0, NWIN, lambda w, __: perm_win_a(w, q), 0)
        plsc.subcore_barrier()
        _export_quarter(spm, sbase, out_a_hbm, rbase, q, kbuf)
        return 0

    lax.fori_loop(1, NQ, a_round, 0)

    iota = lax.iota(jnp.int32, 16)

    def payload_win(w, q):
        base = pl.multiple_of(rbase + w * W, 8)
        pltpu.sync_copy(pos_hbm.at[pl.ds(base, W)], pbuf)
        if not is_pass0:
            pltpu.sync_copy(idx_in_hbm.at[pl.ds(base, W)], ibuf)

        def vreg(jj, _):
            for u_ in range(UNROLL):
                j = jj * UNROLL + u_
                pos = pbuf[pl.ds(j * 16, 16)]
                pbufs[pl.ds(j * 16, 16)] = _quarter_idx(pos, q, sbase)
                if is_pass0:
                    ibuf[pl.ds(j * 16, 16)] = w * W + j * 16 + iota
            return 0

        lax.fori_loop(0, VPW // UNROLL, vreg, 0)
        pltpu.async_copy(ibuf, spm.at[plsc.Indices(pbufs, ignored_value=-1)],
                         sem_out).wait()
        return 0

    def b_round(q, _):
        lax.fori_loop(0, NWIN, lambda w, __: payload_win(w, q), 0)
        plsc.subcore_barrier()
        _export_quarter(spm, sbase, out_b_hbm, rbase, q, kbuf)
        return 0

    lax.fori_loop(0, NQ, b_round, 0)


def _key_out_fn(kk):
    return plsc.bitcast(kk, jnp.int32)


def _prob_out_fn(kk):
    mask = jnp.where(kk >= _SIGN, _ZERO_U, _POSM)
    return plsc.bitcast(kk ^ mask, jnp.int32)


def _run_rows(in_hbm, digit_fn, is_pass0, out_a_fn, out_a, out_b, idx_in,
              pos_hbm, spm, hist, t0, t1, sbuf, ibuf, pbuf, pbufs, kbuf,
              sem_out):
    cid = lax.axis_index("c")
    sid = lax.axis_index("s")
    wid = sid * NC + cid

    def do_row(row_i, _):
        rbase = pl.multiple_of((wid * ROWS_PER_W + row_i) * N, 8)
        _zero_hist(hist)
        _radix_pass(
            in_hbm, digit_fn, is_pass0, out_a_fn, out_a, out_b, idx_in,
            pos_hbm, spm, hist, t0, t1, sbuf, ibuf, pbuf, pbufs, kbuf,
            sem_out, rbase, sid)
        return 0

    lax.fori_loop(0, ROWS_PER_W, do_row, 0)


def _pass0_body(scores, keys_o, idxs_o, pos_o,
                spm, hist, t0, t1, sbuf, ibuf, pbuf, pbufs, kbuf, sem_out):
    _run_rows(scores, _digit_lo, True, _key_out_fn, keys_o, idxs_o, None,
              pos_o, spm, hist, t0, t1, sbuf, ibuf, pbuf, pbufs, kbuf,
              sem_out)


def _pass1_body(keys_i, idxs_i, probs_o, words_o, pos_o,
                spm, hist, t0, t1, sbuf, ibuf, pbuf, pbufs, kbuf, sem_out):
    _run_rows(keys_i, _digit_hi, False, _prob_out_fn, probs_o, words_o,
              idxs_i, pos_o, spm, hist, t0, t1, sbuf, ibuf, pbuf, pbufs,
              kbuf, sem_out)


def _make_kernel(is_pass0):
    mesh = plsc.VectorSubcoreMesh(core_axis_name="c", subcore_axis_name="s")
    return functools.partial(
        pl.kernel,
        out_type=[jax.ShapeDtypeStruct((R * N,), jnp.int32)
                  for _ in range(3)],
        mesh=mesh,
        scratch_types=[
            pltpu.VMEM_SHARED((NS * QH,), jnp.int32),
            pltpu.VMEM((NBINS,), jnp.int32),
            pltpu.VMEM((L1,), jnp.int32),
            pltpu.VMEM((L2,), jnp.int32),
            pltpu.VMEM((W,), jnp.int32),
            pltpu.VMEM((W,), jnp.int32),
            pltpu.VMEM((W,), jnp.int32),
            pltpu.VMEM((W,), jnp.int32),
            pltpu.VMEM((W,), jnp.int32),
            pltpu.SemaphoreType.DMA,
        ],
        compiler_params=pltpu.CompilerParams(needs_layout_passes=False),
    )(_pass0_body if is_pass0 else _pass1_body)


def kernel(scores, k):
    del k
    s_i32 = lax.bitcast_convert_type(scores, jnp.int32).reshape(-1)
    keys, idxs, _ = _make_kernel(True)(s_i32)
    probs_i32, words, _ = _make_kernel(False)(keys, idxs)
    probs = lax.bitcast_convert_type(probs_i32.reshape(R, N), jnp.float32)
    return probs, words.reshape(R, N)


# speedup vs baseline: 8.8838x; 1.0009x over previous
import functools

import jax
import jax.numpy as jnp
import numpy as np
from jax import lax
from jax.experimental import pallas as pl
from jax.experimental.pallas import tpu as pltpu
import jax.experimental.pallas.tpu_sc as plsc

R = 64
N = 100000
NC = 2
NS = 16
NW = NC * NS
ROWS_PER_W = R // NW
W = 2000
NWIN = N // W
VPW = W // 16
UNROLL = 5
NQ = 2
QH = N // NQ
CW = 2000
CWIN = QH // CW
NBINS = 1 << 16
L1 = NBINS // 16
L2 = L1 // 16

_U = jnp.uint32
_SIGN = np.uint32(0x80000000)
_POSM = np.uint32(0x7FFFFFFF)
_ZERO_U = np.uint32(0)


def _key_from_bits(u):
    u = jnp.where(u == _SIGN, _ZERO_U, u)
    mask = jnp.where(u >= _SIGN, _ZERO_U, _POSM)
    return u ^ mask


def _zero_hist(hist):
    zeros = lax.iota(jnp.int32, 16) * 0

    def body(i, _):
        for j in range(16):
            hist[pl.ds((i * 16 + j) * 16, 16)] = zeros
        return 0

    lax.fori_loop(0, L1 // 16, body, 0)


def _prefix_sum(hist, t0, t1):
    iota = lax.iota(jnp.int32, 16)

    def l0(g, _):
        acc = iota * 0
        for j in range(16):
            v = hist[pl.ds((g * 16 + j) * 16, 16)]
            acc = jnp.where(iota == j, jnp.sum(v), acc)
        t0[pl.ds(g * 16, 16)] = acc
        return 0

    lax.fori_loop(0, L1 // 16, l0, 0)

    def l1(g, _):
        acc = iota * 0
        for j in range(16):
            v = t0[pl.ds((g * 16 + j) * 16, 16)]
            acc = jnp.where(iota == j, jnp.sum(v), acc)
        t1[pl.ds(g * 16, 16)] = acc
        return 0

    lax.fori_loop(0, L2 // 16, l1, 0)

    def l2(i, c):
        v = t1[pl.ds(i * 16, 16)]
        s = plsc.cumsum(v)
        t1[pl.ds(i * 16, 16)] = s - v + c
        return c + jnp.sum(v)

    lax.fori_loop(0, L2 // 16, l2, jnp.int32(0))

    def l1b(g, _):
        tv = t1[pl.ds(g * 16, 16)]
        for j in range(16):
            i = g * 16 + j
            v = t0[pl.ds(i * 16, 16)]
            s = plsc.cumsum(v)
            t0[pl.ds(i * 16, 16)] = s - v + tv[j]
        return 0

    lax.fori_loop(0, L2 // 16, l1b, 0)

    def l0b(g, _):
        tv = t0[pl.ds(g * 16, 16)]
        for j in range(16):
            i = g * 16 + j
            v = hist[pl.ds(i * 16, 16)]
            s = plsc.cumsum(v)
            hist[pl.ds(i * 16, 16)] = s - v + tv[j]
        return 0

    lax.fori_loop(0, L1 // 16, l0b, 0)


def _digit_lo(x_i32vec):
    u = plsc.bitcast(x_i32vec, _U)
    kk = _key_from_bits(u)
    return kk, (kk & np.uint32(0xFFFF)).astype(jnp.int32)


def _digit_hi(x_i32vec):
    kk = plsc.bitcast(x_i32vec, _U)
    return kk, (kk >> np.uint32(16)).astype(jnp.int32)


def _export_quarter(spm, sbase, out_hbm, rbase, q, stage):
    def chunk(w, _):
        st = stage.at[pl.ds(0, CW)]
        pltpu.sync_copy(spm.at[pl.ds(sbase + w * CW, CW)], st)
        pltpu.sync_copy(st, out_hbm.at[pl.ds(rbase + q * QH + w * CW, CW)])
        return 0

    lax.fori_loop(0, CWIN, chunk, 0)


def _quarter_idx(pos, q, sbase):
    local = pos - q * QH
    return jnp.where((local >= 0) & (local < QH), local + sbase,
                     jnp.int32(-1))


def _radix_pass(in_hbm, digit_fn, is_pass0, out_a_fn,
                out_a_hbm, out_b_hbm, idx_in_hbm, pos_hbm,
                spm, hist, t0, t1, sbuf, ibuf, pbuf, pbufs, kbuf,
                sem_out, rbase, sid):
    sbase = pl.multiple_of(sid * QH, 8)

    def hist_win(w, _):
        base = pl.multiple_of(rbase + w * W, 8)
        pltpu.sync_copy(in_hbm.at[pl.ds(base, W)], sbuf)

        def vreg(jj, _):
            for u_ in range(UNROLL):
                j = jj * UNROLL + u_
                _, d = digit_fn(sbuf[pl.ds(j * 16, 16)])
                cnt, last = plsc.scan_count(d)
                plsc.addupdate_scatter(hist, [d], cnt, mask=last)
            return 0

        lax.fori_loop(0, VPW // UNROLL, vreg, 0)
        return 0

    with jax.named_scope("histp"):
        lax.fori_loop(0, NWIN, hist_win, 0)
    with jax.named_scope("prefixp"):
        _prefix_sum(hist, t0, t1)

    def perm_win_a0(w, _):
        base = pl.multiple_of(rbase + w * W, 8)
        pltpu.sync_copy(in_hbm.at[pl.ds(base, W)], sbuf)

        def vreg(jj, _):
            for u_ in range(UNROLL):
                j = jj * UNROLL + u_
                kk, d = digit_fn(sbuf[pl.ds(j * 16, 16)])
                cnt, last = plsc.scan_count(d)
                bse = plsc.load_gather(hist, [d])
                pos = bse + cnt - 1
                plsc.store_scatter(hist, [d], pos + 1, mask=last)
                pbuf[pl.ds(j * 16, 16)] = pos
                pbufs[pl.ds(j * 16, 16)] = _quarter_idx(pos, 0, sbase)
                kbuf[pl.ds(j * 16, 16)] = out_a_fn(kk)
            return 0

        lax.fori_loop(0, VPW // UNROLL, vreg, 0)
        pltpu.async_copy(kbuf, spm.at[plsc.Indices(pbufs, ignored_value=-1)],
                         sem_out).wait()
        pltpu.sync_copy(pbuf, pos_hbm.at[pl.ds(base, W)])
        return 0

    with jax.named_scope("a0p"):
        lax.fori_loop(0, NWIN, perm_win_a0, 0)
    plsc.subcore_barrier()
    with jax.named_scope("exp0p"):
        _export_quarter(spm, sbase, out_a_hbm, rbase, 0, kbuf)

    def perm_win_a(w, q):
        base = pl.multiple_of(rbase + w * W, 8)
        pltpu.sync_copy(in_hbm.at[pl.ds(base, W)], sbuf)
        pltpu.sync_copy(pos_hbm.at[pl.ds(base, W)], pbuf)

        def vreg(jj, _):
            for u_ in range(UNROLL):
                j = jj * UNROLL + u_
                kk, _ = digit_fn(sbuf[pl.ds(j * 16, 16)])
                pos = pbuf[pl.ds(j * 16, 16)]
                pbufs[pl.ds(j * 16, 16)] = _quarter_idx(pos, q, sbase)
                kbuf[pl.ds(j * 16, 16)] = out_a_fn(kk)
            return 0

        lax.fori_loop(0, VPW // UNROLL, vreg, 0)
        pltpu.async_copy(kbuf, spm.at[plsc.Indices(pbufs, ignored_value=-1)],
                         sem_out).wait()
        return 0

    def a_round(q, _):
        with jax.named_scope("areplayp"):
            lax.fori_loop(0, NWIN, lambda w, __: perm_win_a(w, q), 0)
        plsc.subcore_barrier()
        with jax.named_scope("expap"):
            _export_quarter(spm, sbase, out_a_hbm, rbase, q, kbuf)
        return 0

    lax.fori_loop(1, NQ, a_round, 0)

    iota = lax.iota(jnp.int32, 16)

    def payload_win(w, q):
        base = pl.multiple_of(rbase + w * W, 8)
        pltpu.sync_copy(pos_hbm.at[pl.ds(base, W)], pbuf)
        if not is_pass0:
            pltpu.sync_copy(idx_in_hbm.at[pl.ds(base, W)], ibuf)

        def vreg(jj, _):
            for u_ in range(UNROLL):
                j = jj * UNROLL + u_
                pos = pbuf[pl.ds(j * 16, 16)]
                pbufs[pl.ds(j * 16, 16)] = _quarter_idx(pos, q, sbase)
                if is_pass0:
                    ibuf[pl.ds(j * 16, 16)] = w * W + j * 16 + iota
            return 0

        lax.fori_loop(0, VPW // UNROLL, vreg, 0)
        pltpu.async_copy(ibuf, spm.at[plsc.Indices(pbufs, ignored_value=-1)],
                         sem_out).wait()
        return 0

    def b_round(q, _):
        with jax.named_scope("bp"):
            lax.fori_loop(0, NWIN, lambda w, __: payload_win(w, q), 0)
        plsc.subcore_barrier()
        with jax.named_scope("expbp"):
            _export_quarter(spm, sbase, out_b_hbm, rbase, q, kbuf)
        return 0

    lax.fori_loop(0, NQ, b_round, 0)


def _key_out_fn(kk):
    return plsc.bitcast(kk, jnp.int32)


def _prob_out_fn(kk):
    mask = jnp.where(kk >= _SIGN, _ZERO_U, _POSM)
    return plsc.bitcast(kk ^ mask, jnp.int32)


def _run_rows(in_hbm, digit_fn, is_pass0, out_a_fn, out_a, out_b, idx_in,
              pos_hbm, spm, hist, t0, t1, sbuf, ibuf, pbuf, pbufs, kbuf,
              sem_out):
    cid = lax.axis_index("c")
    sid = lax.axis_index("s")
    wid = sid * NC + cid

    def do_row(row_i, _):
        rbase = pl.multiple_of((wid * ROWS_PER_W + row_i) * N, 8)
        _zero_hist(hist)
        _radix_pass(
            in_hbm, digit_fn, is_pass0, out_a_fn, out_a, out_b, idx_in,
            pos_hbm, spm, hist, t0, t1, sbuf, ibuf, pbuf, pbufs, kbuf,
            sem_out, rbase, sid)
        return 0

    lax.fori_loop(0, ROWS_PER_W, do_row, 0)


def _pass0_body(scores, keys_o, idxs_o, pos_o,
                spm, hist, t0, t1, sbuf, ibuf, pbuf, pbufs, kbuf, sem_out):
    _run_rows(scores, _digit_lo, True, _key_out_fn, keys_o, idxs_o, None,
              pos_o, spm, hist, t0, t1, sbuf, ibuf, pbuf, pbufs, kbuf,
              sem_out)


def _pass1_body(keys_i, idxs_i, probs_o, words_o, pos_o,
                spm, hist, t0, t1, sbuf, ibuf, pbuf, pbufs, kbuf, sem_out):
    _run_rows(keys_i, _digit_hi, False, _prob_out_fn, probs_o, words_o,
              idxs_i, pos_o, spm, hist, t0, t1, sbuf, ibuf, pbuf, pbufs,
              kbuf, sem_out)


def _make_kernel(is_pass0):
    mesh = plsc.VectorSubcoreMesh(core_axis_name="c", subcore_axis_name="s")
    return functools.partial(
        pl.kernel,
        out_type=[jax.ShapeDtypeStruct((R * N,), jnp.int32)
                  for _ in range(3)],
        mesh=mesh,
        scratch_types=[
            pltpu.VMEM_SHARED((NS * QH,), jnp.int32),
            pltpu.VMEM((NBINS,), jnp.int32),
            pltpu.VMEM((L1,), jnp.int32),
            pltpu.VMEM((L2,), jnp.int32),
            pltpu.VMEM((W,), jnp.int32),
            pltpu.VMEM((W,), jnp.int32),
            pltpu.VMEM((W,), jnp.int32),
            pltpu.VMEM((W,), jnp.int32),
            pltpu.VMEM((W,), jnp.int32),
            pltpu.SemaphoreType.DMA,
        ],
        compiler_params=pltpu.CompilerParams(needs_layout_passes=False),
    )(_pass0_body if is_pass0 else _pass1_body)


def kernel(scores, k):
    del k
    s_i32 = lax.bitcast_convert_type(scores, jnp.int32).reshape(-1)
    keys, idxs, _ = _make_kernel(True)(s_i32)
    probs_i32, words, _ = _make_kernel(False)(keys, idxs)
    probs = lax.bitcast_convert_type(probs_i32.reshape(R, N), jnp.float32)
    return probs, words.reshape(R, N)
